# Initial kernel scaffold; baseline (speedup 1.0000x reference)
import jax
import jax.numpy as jnp
from jax.experimental import pallas as pl


def kernel(q, k, v, pattern_mask):
    raise NotImplementedError("write your pallas kernel here")


# speedup vs baseline: 1345.2920x; 1345.2920x over previous
import functools
import math

import jax
import jax.numpy as jnp
from jax.experimental import pallas as pl


def _masked_attn_kernel(mask_ref, q_ref, k_ref, v_ref, o_ref, *, scale):
    q = q_ref[0]
    k = k_ref[0]
    v = v_ref[0]
    mask = mask_ref[...]

    s = jax.lax.dot_general(
        q, k, (((1,), (1,)), ((), ())), preferred_element_type=jnp.float32
    ) * scale
    s = jnp.where(mask, s, -jnp.inf)
    m = jnp.max(s, axis=1, keepdims=True)
    m = jnp.where(jnp.isfinite(m), m, 0.0)
    e = jnp.where(mask, jnp.exp(s - m), 0.0)
    denom = jnp.sum(e, axis=1, keepdims=True)
    denom = jnp.where(denom == 0.0, 1.0, denom)
    p = e / denom
    o_ref[0] = jax.lax.dot_general(
        p, v, (((1,), (0,)), ((), ())), preferred_element_type=jnp.float32
    )


def kernel(q, k, v, pattern_mask):
    B, S, H, D = q.shape
    G = B * H
    qt = jnp.transpose(q, (0, 2, 1, 3)).reshape(G, S, D)
    kt = jnp.transpose(k, (0, 2, 1, 3)).reshape(G, S, D)
    vt = jnp.transpose(v, (0, 2, 1, 3)).reshape(G, S, D)

    head_spec = pl.BlockSpec((1, S, D), lambda i: (i, 0, 0))
    mask_spec = pl.BlockSpec((S, S), lambda i: (0, 0))

    out = pl.pallas_call(
        functools.partial(_masked_attn_kernel, scale=1.0 / math.sqrt(D)),
        grid=(G,),
        in_specs=[mask_spec, head_spec, head_spec, head_spec],
        out_specs=head_spec,
        out_shape=jax.ShapeDtypeStruct((G, S, D), jnp.float32),
    )(pattern_mask, qt, kt, vt)

    return jnp.transpose(out.reshape(B, H, S, D), (0, 2, 1, 3))


# speedup vs baseline: 1746.1699x; 1.2980x over previous
import functools
import math

import jax
import jax.numpy as jnp
from jax.experimental import pallas as pl


def _masked_attn_kernel(mask_ref, q_ref, k_ref, v_ref, o_ref, *, scale, H, D):
    mask = mask_ref[...]
    for h in range(H):
        sl = slice(h * D, (h + 1) * D)
        q = q_ref[:, sl]
        k = k_ref[:, sl]
        v = v_ref[:, sl]
        s = jax.lax.dot_general(
            q, k, (((1,), (1,)), ((), ())), preferred_element_type=jnp.float32
        ) * scale
        s = jnp.where(mask, s, -jnp.inf)
        m = jnp.max(s, axis=1, keepdims=True)
        m = jnp.where(jnp.isfinite(m), m, 0.0)
        e = jnp.where(mask, jnp.exp(s - m), 0.0)
        denom = jnp.sum(e, axis=1, keepdims=True)
        denom = jnp.where(denom == 0.0, 1.0, denom)
        p = e / denom
        o_ref[:, sl] = jax.lax.dot_general(
            p, v, (((1,), (0,)), ((), ())), preferred_element_type=jnp.float32
        )


def kernel(q, k, v, pattern_mask):
    B, S, H, D = q.shape
    q2 = q.reshape(B * S, H * D)
    k2 = k.reshape(B * S, H * D)
    v2 = v.reshape(B * S, H * D)

    out = pl.pallas_call(
        functools.partial(
            _masked_attn_kernel, scale=1.0 / math.sqrt(D), H=H, D=D
        ),
        out_shape=jax.ShapeDtypeStruct((B * S, H * D), jnp.float32),
    )(pattern_mask, q2, k2, v2)

    return out.reshape(B, S, H, D)


# speedup vs baseline: 1846.5949x; 1.0575x over previous
import functools
import math

import jax
import jax.numpy as jnp
from jax.experimental import pallas as pl


def _masked_attn_kernel(mask_ref, q_ref, k_ref, v_ref, o_ref, *, scale, H, D):
    mask = mask_ref[...]
    for h in range(H):
        sl = slice(h * D, (h + 1) * D)
        q = q_ref[:, sl].astype(jnp.bfloat16)
        k = k_ref[:, sl].astype(jnp.bfloat16)
        v = v_ref[:, sl].astype(jnp.bfloat16)
        s = jax.lax.dot_general(
            q, k, (((1,), (1,)), ((), ())), preferred_element_type=jnp.float32
        ) * scale
        s = jnp.where(mask, s, -1e30)
        m = jnp.max(s, axis=1, keepdims=True)
        m = jnp.maximum(m, -1e29)
        e = jnp.exp(s - m)
        denom = jnp.sum(e, axis=1, keepdims=True)
        p = e * (1.0 / jnp.maximum(denom, 1.0))
        o_ref[:, sl] = jax.lax.dot_general(
            p.astype(jnp.bfloat16), v, (((1,), (0,)), ((), ())),
            preferred_element_type=jnp.float32,
        )


def kernel(q, k, v, pattern_mask):
    B, S, H, D = q.shape
    q2 = q.reshape(B * S, H * D)
    k2 = k.reshape(B * S, H * D)
    v2 = v.reshape(B * S, H * D)

    out = pl.pallas_call(
        functools.partial(
            _masked_attn_kernel, scale=1.0 / math.sqrt(D), H=H, D=D
        ),
        out_shape=jax.ShapeDtypeStruct((B * S, H * D), jnp.float32),
    )(pattern_mask, q2, k2, v2)

    return out.reshape(B, S, H, D)


# speedup vs baseline: 1916.9258x; 1.0381x over previous
import functools
import math

import jax
import jax.numpy as jnp
from jax.experimental import pallas as pl


def _masked_attn_kernel(mask_ref, q_ref, k_ref, v_ref, o_ref, *, scale, H, D):
    mask = mask_ref[...]
    for h in range(H):
        sl = slice(h * D, (h + 1) * D)
        q = (q_ref[:, sl] * scale).astype(jnp.bfloat16)
        k = k_ref[:, sl].astype(jnp.bfloat16)
        v = v_ref[:, sl].astype(jnp.bfloat16)
        s = jax.lax.dot_general(
            q, k, (((1,), (1,)), ((), ())), preferred_element_type=jnp.float32
        )
        s = jnp.where(mask, s, -1e30)
        m = jnp.max(s, axis=1, keepdims=True)
        m = jnp.maximum(m, -1e29)
        e = jnp.exp(s - m)
        denom = jnp.sum(e, axis=1, keepdims=True)
        r = 1.0 / jnp.maximum(denom, 1.0)
        o = jax.lax.dot_general(
            e.astype(jnp.bfloat16), v, (((1,), (0,)), ((), ())),
            preferred_element_type=jnp.float32,
        )
        o_ref[:, sl] = o * r


def kernel(q, k, v, pattern_mask):
    B, S, H, D = q.shape
    q2 = q.reshape(B * S, H * D)
    k2 = k.reshape(B * S, H * D)
    v2 = v.reshape(B * S, H * D)

    out = pl.pallas_call(
        functools.partial(
            _masked_attn_kernel, scale=1.0 / math.sqrt(D), H=H, D=D
        ),
        out_shape=jax.ShapeDtypeStruct((B * S, H * D), jnp.float32),
    )(pattern_mask, q2, k2, v2)

    return out.reshape(B, S, H, D)
